# Initial kernel scaffold; baseline (speedup 1.0000x reference)
import jax
import jax.numpy as jnp
from jax.experimental import pallas as pl


def kernel(x, perms):
    raise NotImplementedError("write your pallas kernel here")


# speedup vs baseline: 1.1521x; 1.1521x over previous
import jax
import jax.numpy as jnp
from jax import lax
from jax.experimental import pallas as pl
from jax.experimental.pallas import tpu as pltpu
from jax.experimental.pallas import tpu_sc as plsc

_NC = 2
_NS = 16


def _make_sc_kernel(B, C, HW):
  n_tile = HW // _NS
  b_per_core = B // _NC
  mesh = plsc.VectorSubcoreMesh(core_axis_name="c", subcore_axis_name="s")

  def body(x_hbm, perm_hbm, out_hbm, idx_v, out_v, img_sh):
    cid = lax.axis_index("c")
    sid = lax.axis_index("s")
    base = sid * n_tile

    for ch in range(C):
      pltpu.sync_copy(perm_hbm.at[pl.ds(ch * HW + base, n_tile)], idx_v)

      def batch_step(bi, carry):
        b = cid * b_per_core + bi
        off = b * (C * HW) + ch * HW + base
        pltpu.sync_copy(x_hbm.at[pl.ds(off, n_tile)],
                        img_sh.at[pl.ds(base, n_tile)])
        plsc.subcore_barrier()
        pltpu.sync_copy(img_sh.at[idx_v], out_v)
        pltpu.sync_copy(out_v, out_hbm.at[pl.ds(off, n_tile)])
        plsc.subcore_barrier()
        return carry

      lax.fori_loop(0, b_per_core, batch_step, 0)

  return pl.kernel(
      body,
      out_type=jax.ShapeDtypeStruct((B * C * HW,), jnp.float32),
      mesh=mesh,
      scratch_types=[
          pltpu.VMEM((n_tile,), jnp.int32),
          pltpu.VMEM((n_tile,), jnp.float32),
          pltpu.VMEM_SHARED((HW,), jnp.float32),
      ],
  )


def kernel(x, perms):
  B, C, H, W = x.shape
  HW = H * W
  x_flat = x.reshape(B * C * HW)
  perms_i32 = perms.astype(jnp.int32).reshape(C * HW)
  out = _make_sc_kernel(B, C, HW)(x_flat, perms_i32)
  return out.reshape(B, C, H, W)


# speedup vs baseline: 1.3713x; 1.1903x over previous
import jax
import jax.numpy as jnp
from jax import lax
from jax.experimental import pallas as pl
from jax.experimental.pallas import tpu as pltpu
from jax.experimental.pallas import tpu_sc as plsc

_NC = 2
_NS = 16


def _make_sc_kernel(B, C, HW):
  n_tile = HW // _NS
  b_per_core = B // _NC
  mesh = plsc.VectorSubcoreMesh(core_axis_name="c", subcore_axis_name="s")

  def body(x_hbm, perm_hbm, out_hbm, idx_v, out_v, img0, img1, sem0, sem1):
    cid = lax.axis_index("c")
    sid = lax.axis_index("s")
    base = sid * n_tile

    def stage_copy(ch, bi, img, sem):
      b = cid * b_per_core + bi
      off = b * (C * HW) + ch * HW + base
      return pltpu.make_async_copy(
          x_hbm.at[pl.ds(off, n_tile)],
          img.at[pl.ds(base, n_tile)], sem)

    def gather_write(ch, bi, img):
      b = cid * b_per_core + bi
      off = b * (C * HW) + ch * HW + base
      pltpu.sync_copy(img.at[idx_v], out_v)
      pltpu.sync_copy(out_v, out_hbm.at[pl.ds(off, n_tile)])

    for ch in range(C):
      pltpu.sync_copy(perm_hbm.at[pl.ds(ch * HW + base, n_tile)], idx_v)
      stage_copy(ch, 0, img0, sem0).start()
      stage_copy(ch, 1, img1, sem1).start()

      def batch_step(i, carry):
        bi = 2 * i
        stage_copy(ch, bi, img0, sem0).wait()
        plsc.subcore_barrier()
        gather_write(ch, bi, img0)
        plsc.subcore_barrier()

        @pl.when(bi + 2 < b_per_core)
        def _():
          stage_copy(ch, bi + 2, img0, sem0).start()

        stage_copy(ch, bi + 1, img1, sem1).wait()
        plsc.subcore_barrier()
        gather_write(ch, bi + 1, img1)
        plsc.subcore_barrier()

        @pl.when(bi + 3 < b_per_core)
        def _():
          stage_copy(ch, bi + 3, img1, sem1).start()

        return carry

      lax.fori_loop(0, b_per_core // 2, batch_step, 0)

  return pl.kernel(
      body,
      out_type=jax.ShapeDtypeStruct((B * C * HW,), jnp.float32),
      mesh=mesh,
      scratch_types=[
          pltpu.VMEM((n_tile,), jnp.int32),
          pltpu.VMEM((n_tile,), jnp.float32),
          pltpu.VMEM_SHARED((HW,), jnp.float32),
          pltpu.VMEM_SHARED((HW,), jnp.float32),
          pltpu.SemaphoreType.DMA,
          pltpu.SemaphoreType.DMA,
      ],
  )


def kernel(x, perms):
  B, C, H, W = x.shape
  HW = H * W
  x_flat = x.reshape(B * C * HW)
  perms_i32 = perms.astype(jnp.int32).reshape(C * HW)
  out = _make_sc_kernel(B, C, HW)(x_flat, perms_i32)
  return out.reshape(B, C, H, W)
